# Initial kernel scaffold; baseline (speedup 1.0000x reference)
import jax
import jax.numpy as jnp
from jax.experimental import pallas as pl


def kernel(x, Wr, W1, b1, W2, b2, gamma, beta):
    raise NotImplementedError("write your pallas kernel here")


# speedup vs baseline: 2.2443x; 2.2443x over previous
import functools

import jax
import jax.numpy as jnp
from jax.experimental import pallas as pl
from jax.experimental.pallas import tpu as pltpu

D_MODEL = 768
D_FF = 3072
E = 8
K = 2
T = 256


def _router_kernel(x_ref, wr_ref, gd_ref, rankx_ref, nsel_ref):
    x = x_ref[...]
    logits = jnp.dot(x, wr_ref[...], preferred_element_type=jnp.float32)
    n = logits.shape[0]
    iota_e = jax.lax.broadcasted_iota(jnp.int32, (1, E), 1)
    big = jnp.int32(E)

    m1 = jnp.max(logits, axis=1, keepdims=True)
    eq1 = logits == m1
    i1 = jnp.min(jnp.where(eq1, iota_e, big), axis=1, keepdims=True)
    sel1 = (iota_e == i1)
    logits2 = jnp.where(sel1, -jnp.inf, logits)
    m2 = jnp.max(logits2, axis=1, keepdims=True)
    eq2 = logits2 == m2
    i2 = jnp.min(jnp.where(eq2, iota_e, big), axis=1, keepdims=True)
    sel2 = (iota_e == i2)

    z = jnp.exp(m2 - m1)
    p1 = 1.0 / (1.0 + z)
    p2 = z / (1.0 + z)
    gd_ref[...] = jnp.where(sel1, p1, 0.0) + jnp.where(sel2, p2, 0.0)

    sel = (sel1 | sel2).astype(jnp.float32)

    c = 256
    iota_r = jax.lax.broadcasted_iota(jnp.int32, (c, c), 0)
    iota_c = jax.lax.broadcasted_iota(jnp.int32, (c, c), 1)
    ltri = (iota_r > iota_c).astype(jnp.float32)
    running = jnp.zeros((1, E), jnp.float32)
    chunks = []
    for ci in range(n // c):
        s = jax.lax.slice(sel, (ci * c, 0), (ci * c + c, E))
        ex = jnp.dot(ltri, s, preferred_element_type=jnp.float32) + running
        chunks.append(jnp.where(s > 0, ex, -1.0))
        running = running + jnp.sum(s, axis=0, keepdims=True)
    rankx = jnp.concatenate(chunks, axis=0)
    rankx_ref[...] = rankx.T.reshape(E, 1, n)
    nsel_ref[...] = running.astype(jnp.int32)


def _ffn_kernel(nsel_ref, x_ref, rankx_ref, gd_ref, w1_ref, b1_ref, w2_ref,
                b2_ref, g_ref, b_ref, out_ref):
    e = pl.program_id(0)
    rb = pl.program_id(1)

    @pl.when(jnp.logical_and(e == 0, rb == 0))
    def _init():
        out_ref[...] = jnp.zeros_like(out_ref)

    nsel_e = nsel_ref[0, e]
    r0 = rb * T

    @pl.when(r0 < nsel_e)
    def _active():
        n = x_ref.shape[0]
        rows = r0 + jax.lax.broadcasted_iota(jnp.int32, (T, 1), 0)
        rank_e = rankx_ref[0]
        onehot = (rank_e == rows.astype(jnp.float32)).astype(jnp.float32)
        xg = jnp.dot(onehot, x_ref[...], preferred_element_type=jnp.float32)
        h1 = jnp.dot(xg, w1_ref[0], preferred_element_type=jnp.float32)
        h1 = jnp.maximum(h1 + b1_ref[0], 0.0)
        h2 = jnp.dot(h1, w2_ref[0], preferred_element_type=jnp.float32)
        h = xg + h2 + b2_ref[0]
        mu = jnp.mean(h, axis=-1, keepdims=True)
        var = jnp.mean((h - mu) ** 2, axis=-1, keepdims=True)
        y = (h - mu) / jnp.sqrt(var + 1e-6) * g_ref[0] + b_ref[0]

        iota_e = jax.lax.broadcasted_iota(jnp.int32, (T, E), 1)
        gcol = jnp.sum(jnp.where(iota_e == e, gd_ref[...], 0.0), axis=1,
                       keepdims=True)
        mask = (rows < nsel_e).astype(jnp.float32)
        out_ref[pl.ds(r0, T), :] += y * (gcol * mask)


@jax.jit
def kernel(x, Wr, W1, b1, W2, b2, gamma, beta):
    B, S, D = x.shape
    N = B * S
    xf = x.reshape(N, D)

    gd, rankx, nsel = pl.pallas_call(
        _router_kernel,
        out_shape=(
            jax.ShapeDtypeStruct((N, E), jnp.float32),
            jax.ShapeDtypeStruct((E, 1, N), jnp.float32),
            jax.ShapeDtypeStruct((1, E), jnp.int32),
        ),
    )(xf, Wr)

    out = pl.pallas_call(
        _ffn_kernel,
        grid=(E, N // T),
        in_specs=[
            pl.BlockSpec(memory_space=pltpu.SMEM),
            pl.BlockSpec((N, D), lambda e, rb: (0, 0)),
            pl.BlockSpec((1, 1, N), lambda e, rb: (e, 0, 0)),
            pl.BlockSpec((T, E), lambda e, rb: (rb, 0)),
            pl.BlockSpec((1, D, D_FF), lambda e, rb: (e, 0, 0)),
            pl.BlockSpec((1, 1, D_FF), lambda e, rb: (e, 0, 0)),
            pl.BlockSpec((1, D_FF, D), lambda e, rb: (e, 0, 0)),
            pl.BlockSpec((1, 1, D), lambda e, rb: (e, 0, 0)),
            pl.BlockSpec((1, 1, D), lambda e, rb: (e, 0, 0)),
            pl.BlockSpec((1, 1, D), lambda e, rb: (e, 0, 0)),
        ],
        out_specs=pl.BlockSpec((N, D), lambda e, rb: (0, 0)),
        out_shape=jax.ShapeDtypeStruct((N, D), jnp.float32),
    )(nsel, xf, rankx, gd, W1, b1.reshape(E, 1, D_FF), W2,
      b2.reshape(E, 1, D), gamma.reshape(E, 1, D), beta.reshape(E, 1, D))

    return out.reshape(B, S, D)
